# Initial kernel scaffold; baseline (speedup 1.0000x reference)
import jax
import jax.numpy as jnp
from jax.experimental import pallas as pl


def kernel(indices, W):
    raise NotImplementedError("write your pallas kernel here")


# speedup vs baseline: 9.2471x; 9.2471x over previous
import functools

import jax
import jax.numpy as jnp
from jax import lax
from jax.experimental import pallas as pl
from jax.experimental.pallas import tpu as pltpu
from jax.experimental.pallas import tpu_sc as plsc

BATCH = 4096
HIST = 200
D = 128
B = BATCH * HIST

NC = 2
NS = 16
NW = NC * NS
BPW = B // NW
CH = 128
NCH = BPW // CH
NBUF = 4
NG = NCH // NBUF


def _body(idx_hbm, table_hbm, out_hbm, idx_v, rows_v, gsem, ssem):
    c = lax.axis_index("c")
    s = lax.axis_index("s")
    wid = s * NC + c
    chunk0 = wid * NCH

    pltpu.sync_copy(idx_hbm.at[pl.ds(chunk0, NCH)], idx_v)

    def start_gather(j, b):
        pltpu.async_copy(table_hbm.at[idx_v.at[j]], rows_v.at[b], gsem.at[b])

    def wait_gather(b):
        pltpu.make_async_copy(table_hbm.at[idx_v.at[0]], rows_v.at[b],
                              gsem.at[b]).wait()

    def start_scatter(j, b):
        row0 = (chunk0 + j) * CH
        pltpu.async_copy(rows_v.at[b], out_hbm.at[pl.ds(row0, CH)], ssem.at[b])

    def wait_scatter(b):
        pltpu.make_async_copy(rows_v.at[b], out_hbm.at[pl.ds(0, CH)],
                              ssem.at[b]).wait()

    for b in range(NBUF):
        start_gather(b, b)

    def group(g, carry):
        for b in range(NBUF):
            j = g * NBUF + b
            wait_gather(b)
            start_scatter(j, b)
            wait_scatter(b)
            start_gather(j + NBUF, b)
        return carry

    lax.fori_loop(0, NG - 1, group, 0)

    for b in range(NBUF):
        j = (NG - 1) * NBUF + b
        wait_gather(b)
        start_scatter(j, b)
    for b in range(NBUF):
        wait_scatter(b)


@jax.jit
def _gather(idx2d, W):
    mesh = plsc.VectorSubcoreMesh(core_axis_name="c", subcore_axis_name="s")
    fn = pl.kernel(
        _body,
        out_type=jax.ShapeDtypeStruct((B, D), jnp.float32),
        mesh=mesh,
        scratch_types=[
            pltpu.VMEM((NCH, CH), jnp.int32),
            pltpu.VMEM((NBUF, CH, D), jnp.float32),
            pltpu.SemaphoreType.DMA((NBUF,)),
            pltpu.SemaphoreType.DMA((NBUF,)),
        ],
    )
    return fn(idx2d, W)


def kernel(indices, W):
    idx2d = indices.astype(jnp.int32).reshape(B // CH, CH)
    out = _gather(idx2d, W)
    return out.reshape(BATCH, HIST, D)


# speedup vs baseline: 9.2530x; 1.0006x over previous
import functools

import jax
import jax.numpy as jnp
from jax import lax
from jax.experimental import pallas as pl
from jax.experimental.pallas import tpu as pltpu
from jax.experimental.pallas import tpu_sc as plsc

BATCH = 4096
HIST = 200
D = 128
B = BATCH * HIST

NC = 2
NS = 16
NW = NC * NS
BPW = B // NW
CH = 128
NCH = BPW // CH
NBUF = 5
NG = NCH // NBUF


def _body(idx_hbm, table_hbm, out_hbm, idx_v, rows_v, gsem, ssem):
    c = lax.axis_index("c")
    s = lax.axis_index("s")
    wid = s * NC + c
    chunk0 = wid * NCH

    pltpu.sync_copy(idx_hbm.at[pl.ds(chunk0, NCH)], idx_v)

    def start_gather(j, b):
        pltpu.async_copy(table_hbm.at[idx_v.at[j]], rows_v.at[b], gsem.at[b])

    def wait_gather(b):
        pltpu.make_async_copy(table_hbm.at[idx_v.at[0]], rows_v.at[b],
                              gsem.at[b]).wait()

    def start_scatter(j, b):
        row0 = (chunk0 + j) * CH
        pltpu.async_copy(rows_v.at[b], out_hbm.at[pl.ds(row0, CH)], ssem.at[b])

    def wait_scatter(b):
        pltpu.make_async_copy(rows_v.at[b], out_hbm.at[pl.ds(0, CH)],
                              ssem.at[b]).wait()

    for b in range(NBUF):
        start_gather(b, b)

    def group(g, carry):
        for b in range(NBUF):
            j = g * NBUF + b
            wait_gather(b)
            start_scatter(j, b)
            wait_scatter(b)
            start_gather(j + NBUF, b)
        return carry

    lax.fori_loop(0, NG - 1, group, 0)

    for b in range(NBUF):
        j = (NG - 1) * NBUF + b
        wait_gather(b)
        start_scatter(j, b)
    for b in range(NBUF):
        wait_scatter(b)


@jax.jit
def _gather(idx2d, W):
    mesh = plsc.VectorSubcoreMesh(core_axis_name="c", subcore_axis_name="s")
    fn = pl.kernel(
        _body,
        out_type=jax.ShapeDtypeStruct((B, D), jnp.float32),
        mesh=mesh,
        scratch_types=[
            pltpu.VMEM((NCH, CH), jnp.int32),
            pltpu.VMEM((NBUF, CH, D), jnp.float32),
            pltpu.SemaphoreType.DMA((NBUF,)),
            pltpu.SemaphoreType.DMA((NBUF,)),
        ],
    )
    return fn(idx2d, W)


def kernel(indices, W):
    idx2d = indices.astype(jnp.int32).reshape(B // CH, CH)
    out = _gather(idx2d, W)
    return out.reshape(BATCH, HIST, D)
